# baseline (speedup 1.0000x reference)
import jax
import jax.numpy as jnp
from jax.experimental import pallas as pl
from jax.experimental.pallas import tpu as pltpu

_ROWS = 16384
_D = 1024
_BLK = 2048
_GRID = _ROWS // _BLK
_H = _D // 2


def _body(xa_ref, xb_ref, yt_ref, yp_ref, out_ref, acc_ref):
    i = pl.program_id(0)

    @pl.when(i == 0)
    def _init():
        acc_ref[0] = 0.0
        acc_ref[1] = 0.0

    rowsum = jnp.sum(xa_ref[...], axis=1) + jnp.sum(xb_ref[...], axis=1)
    mask = (rowsum > 0.0).astype(jnp.float32)
    yt = yt_ref[0, 0, :]
    yp = yp_ref[0, 0, :]
    correct = jnp.where(
        (yp > 0.5) & (yt == 1.0) | (yp <= 0.5) & (yt == 0.0), 1.0, 0.0)
    acc_ref[0] += jnp.sum(correct * mask)
    acc_ref[1] += jnp.sum(mask)

    @pl.when(i == _GRID - 1)
    def _fin():
        out_ref[...] = jnp.full((1, 1), acc_ref[0] / acc_ref[1], jnp.float32)


def kernel(input_data, y_true, y_pred):
    x = input_data.reshape(_ROWS, _D)
    yt = y_true.reshape(_GRID, 1, _BLK)
    yp = y_pred.reshape(_GRID, 1, _BLK)
    out = pl.pallas_call(
        _body,
        grid=(_GRID,),
        in_specs=[
            pl.BlockSpec((_BLK, _H), lambda i: (i, 0)),
            pl.BlockSpec((_BLK, _H), lambda i: (i, 1)),
            pl.BlockSpec((1, 1, _BLK), lambda i: (i, 0, 0)),
            pl.BlockSpec((1, 1, _BLK), lambda i: (i, 0, 0)),
        ],
        out_specs=pl.BlockSpec((1, 1), lambda i: (0, 0)),
        out_shape=jax.ShapeDtypeStruct((1, 1), jnp.float32),
        scratch_shapes=[pltpu.SMEM((2,), jnp.float32)],
    )(x, x, yt, yp)
    return out[0, 0]


# speedup vs baseline: 1.0080x; 1.0080x over previous
import jax
import jax.numpy as jnp
from jax.experimental import pallas as pl
from jax.experimental.pallas import tpu as pltpu

_ROWS = 16384
_D = 1024
_BLK = 2048
_GRID = _ROWS // _BLK


def _body(x_ref, yt_ref, yp_ref, out_ref, acc_ref):
    i = pl.program_id(0)

    @pl.when(i == 0)
    def _init():
        acc_ref[0] = 0.0
        acc_ref[1] = 0.0

    rowsum = jnp.sum(x_ref[...], axis=1)
    mask = (rowsum > 0.0).astype(jnp.float32)
    yt = yt_ref[0, 0, :]
    yp = yp_ref[0, 0, :]
    correct = jnp.where(
        (yp > 0.5) & (yt == 1.0) | (yp <= 0.5) & (yt == 0.0), 1.0, 0.0)
    acc_ref[0] += jnp.sum(correct * mask)
    acc_ref[1] += jnp.sum(mask)

    @pl.when(i == _GRID - 1)
    def _fin():
        out_ref[...] = jnp.full((1, 1), acc_ref[0] / acc_ref[1], jnp.float32)


def kernel(input_data, y_true, y_pred):
    x = input_data.reshape(_ROWS, _D)
    yt = y_true.reshape(_GRID, 1, _BLK)
    yp = y_pred.reshape(_GRID, 1, _BLK)
    out = pl.pallas_call(
        _body,
        grid=(_GRID,),
        in_specs=[
            pl.BlockSpec((_BLK, _D), lambda i: (i, 0)),
            pl.BlockSpec((1, 1, _BLK), lambda i: (i, 0, 0)),
            pl.BlockSpec((1, 1, _BLK), lambda i: (i, 0, 0)),
        ],
        out_specs=pl.BlockSpec((1, 1), lambda i: (0, 0)),
        out_shape=jax.ShapeDtypeStruct((1, 1), jnp.float32),
        scratch_shapes=[pltpu.SMEM((2,), jnp.float32)],
    )(x, yt, yp)
    return out[0, 0]


# speedup vs baseline: 1.0110x; 1.0030x over previous
import jax
import jax.numpy as jnp
from jax.experimental import pallas as pl
from jax.experimental.pallas import tpu as pltpu

_ROWS = 16384
_D = 1024
_BLK = 2048
_GRID = _ROWS // _BLK


def _body(x_ref, yt_ref, yp_ref, out_ref, acc_ref):
    i = pl.program_id(0)

    @pl.when(i == 0)
    def _init():
        acc_ref[0] = 0.0
        acc_ref[1] = 0.0

    ones = jnp.ones((_D,), jnp.float32)
    rowsum = jax.lax.dot_general(
        x_ref[...], ones, (((1,), (0,)), ((), ())),
        preferred_element_type=jnp.float32)
    mask = (rowsum > 0.0).astype(jnp.float32)
    yt = yt_ref[0, 0, :]
    yp = yp_ref[0, 0, :]
    correct = jnp.where(
        (yp > 0.5) & (yt == 1.0) | (yp <= 0.5) & (yt == 0.0), 1.0, 0.0)
    acc_ref[0] += jnp.sum(correct * mask)
    acc_ref[1] += jnp.sum(mask)

    @pl.when(i == _GRID - 1)
    def _fin():
        out_ref[...] = jnp.full((1, 1), acc_ref[0] / acc_ref[1], jnp.float32)


def kernel(input_data, y_true, y_pred):
    x = input_data.reshape(_ROWS, _D)
    yt = y_true.reshape(_GRID, 1, _BLK)
    yp = y_pred.reshape(_GRID, 1, _BLK)
    out = pl.pallas_call(
        _body,
        grid=(_GRID,),
        in_specs=[
            pl.BlockSpec((_BLK, _D), lambda i: (i, 0)),
            pl.BlockSpec((1, 1, _BLK), lambda i: (i, 0, 0)),
            pl.BlockSpec((1, 1, _BLK), lambda i: (i, 0, 0)),
        ],
        out_specs=pl.BlockSpec((1, 1), lambda i: (0, 0)),
        out_shape=jax.ShapeDtypeStruct((1, 1), jnp.float32),
        scratch_shapes=[pltpu.SMEM((2,), jnp.float32)],
    )(x, yt, yp)
    return out[0, 0]


# speedup vs baseline: 1.0143x; 1.0033x over previous
import jax
import jax.numpy as jnp
from jax.experimental import pallas as pl
from jax.experimental.pallas import tpu as pltpu

_ROWS = 16384
_D = 1024
_BLK = 2048
_GRID = _ROWS // _BLK


def _body(x_ref, yt_ref, yp_ref, out_ref, acc_ref):
    i = pl.program_id(0)

    @pl.when(i == 0)
    def _init():
        acc_ref[0] = 0.0
        acc_ref[1] = 0.0

    rowsum = jnp.sum(x_ref[...], axis=1)
    mask = (rowsum > 0.0).astype(jnp.float32)
    yt = yt_ref[0, 0, :]
    yp = yp_ref[0, 0, :]
    correct = jnp.where(
        (yp > 0.5) & (yt == 1.0) | (yp <= 0.5) & (yt == 0.0), 1.0, 0.0)
    acc_ref[0] += jnp.sum(correct * mask)
    acc_ref[1] += jnp.sum(mask)

    @pl.when(i == _GRID - 1)
    def _fin():
        out_ref[...] = jnp.full((1, 1), acc_ref[0] / acc_ref[1], jnp.float32)


def kernel(input_data, y_true, y_pred):
    x = input_data.reshape(_ROWS, _D)
    yt = y_true.reshape(_GRID, 1, _BLK)
    yp = y_pred.reshape(_GRID, 1, _BLK)
    out = pl.pallas_call(
        _body,
        grid=(_GRID,),
        in_specs=[
            pl.BlockSpec((_BLK, _D), lambda i: (i, 0)),
            pl.BlockSpec((1, 1, _BLK), lambda i: (i, 0, 0)),
            pl.BlockSpec((1, 1, _BLK), lambda i: (i, 0, 0)),
        ],
        out_specs=pl.BlockSpec((1, 1), lambda i: (0, 0)),
        out_shape=jax.ShapeDtypeStruct((1, 1), jnp.float32),
        scratch_shapes=[pltpu.SMEM((2,), jnp.float32)],
    )(x, yt, yp)
    return out[0, 0]
